# baseline (speedup 1.0000x reference)
import functools

import jax
import jax.numpy as jnp
from jax import lax
from jax.experimental import pallas as pl
from jax.experimental.pallas import tpu as pltpu
from jax.experimental.pallas import tpu_sc as plsc

USERS = 100000
ITEMS = 100000
SESSIONS = 100000
D = 128
B = 4096
L = 200
AVG_RATING = 3.5
CURRENT_DAY = 17990.0

LJ = 100
NCHUNK = 2
CHUNK_OFF = (0, 100)
NACC = D // 16


def _build_sc_call(nc, ns):
    nw = nc * ns
    per_w = B // nw
    mesh = plsc.VectorSubcoreMesh(core_axis_name="c", subcore_axis_name="s")

    @functools.partial(
        pl.kernel,
        mesh=mesh,
        out_type=jax.ShapeDtypeStruct((B,), jnp.float32),
        compiler_params=pltpu.CompilerParams(needs_layout_passes=False,
                                             use_tc_tiling_on_sc=False),
        scratch_types=[
            pltpu.VMEM((per_w, NCHUNK, LJ), jnp.int32),
            pltpu.VMEM((L,), jnp.int32),
            pltpu.VMEM((L,), jnp.int32),
            pltpu.VMEM((per_w, D), jnp.float32),
            pltpu.VMEM((per_w, D), jnp.float32),
            pltpu.VMEM((L, D // 2), jnp.int32),
            pltpu.VMEM((L, D // 2), jnp.int32),
            pltpu.VMEM((per_w,), jnp.int32),
            pltpu.VMEM((per_w,), jnp.int32),
            pltpu.VMEM((per_w,), jnp.int32),
            pltpu.VMEM((per_w,), jnp.float32),
            pltpu.VMEM((per_w,), jnp.float32),
            pltpu.VMEM((per_w,), jnp.float32),
            pltpu.VMEM((per_w,), jnp.float32),
            pltpu.VMEM((per_w,), jnp.float32),
            pltpu.SemaphoreType.DMA,
            pltpu.SemaphoreType.DMA,
            pltpu.SemaphoreType.DMA,
        ],
    )
    def sc_call(uid_hbm, iid_hbm, sid_hbm, days_hbm, uemb_hbm, iemb_hbm,
                iembbf_hbm, ubias_hbm, ibias_hbm, sbias_hbm, udec_hbm,
                out_hbm,
                ids_v, d0, d1, u_rows, i_rows, e0, e1, uid_v, iid_v, sidx_v,
                dec_v, ub_v, ib_v, sb_v, out_v, sem0, sem1, semm):
        wid = lax.axis_index("s") * nc + lax.axis_index("c")
        base = wid * per_w
        lane = lax.iota(jnp.int32, 16)
        idx_even = (2 * lane) & 15
        idx_odd = (2 * lane + 1) & 15
        himask = jnp.full((16,), jnp.int32(-65536))
        splats = [jnp.full((16,), jnp.int32(k)) for k in range(16)]

        pltpu.sync_copy(uid_hbm.at[pl.ds(base, per_w)], uid_v)
        pltpu.sync_copy(iid_hbm.at[pl.ds(base, per_w)], iid_v)
        pltpu.sync_copy(sid_hbm.at[pl.ds(base, per_w)], ids_v)

        cp_u = pltpu.make_async_copy(uemb_hbm.at[uid_v], u_rows, semm)
        cp_d = pltpu.make_async_copy(udec_hbm.at[uid_v], dec_v, semm)
        cp_ub = pltpu.make_async_copy(ubias_hbm.at[uid_v], ub_v, semm)
        cp_ib = pltpu.make_async_copy(ibias_hbm.at[iid_v], ib_v, semm)
        for cp in (cp_u, cp_d, cp_ub, cp_ib):
            cp.start()
        for cp in (cp_u, cp_d, cp_ub, cp_ib):
            cp.wait()

        def start_gather(b, ebuf, dbuf, sem):
            for j in range(NCHUNK):
                pltpu.make_async_copy(
                    iembbf_hbm.at[ids_v.at[b, j]],
                    ebuf.at[pl.ds(CHUNK_OFF[j], LJ)], sem).start()
            pltpu.make_async_copy(days_hbm.at[base + b], dbuf, sem).start()

        def wait_gather(b, ebuf, dbuf, sem):
            for j in range(NCHUNK):
                pltpu.make_async_copy(
                    iembbf_hbm.at[ids_v.at[b, j]],
                    ebuf.at[pl.ds(CHUNK_OFF[j], LJ)], sem).wait()
            pltpu.make_async_copy(days_hbm.at[base + b], dbuf, sem).wait()

        def vsum(v):
            return plsc.cumsum(v)[15]

        def row_dot(row_a, b, accs):
            s = jnp.zeros((16,), jnp.float32)
            for c in range(NACC):
                s = s + accs[c] * row_a[b, pl.ds(c * 16, 16)]
            return vsum(s)

        def compute_b(b, ebuf, dbuf):
            grp = (b // 16) * 16
            rchunk = dec_v[pl.ds(grp, 16)]
            rate = vsum(jnp.where(lane == b - grp, rchunk, 0.0))
            rate = jnp.maximum(rate, 0.0)

            def accum_lanes(w, l0, accs, lanes):
                for k in lanes:
                    wv = jnp.take(w, splats[k], mode="fill")
                    l = l0 + k
                    new = list(accs)
                    for c in range(NACC // 2):
                        pair = ebuf[l, pl.ds(c * 16, 16)]
                        ev = plsc.bitcast(jnp.left_shift(pair, 16),
                                          jnp.float32)
                        od = plsc.bitcast(pair & himask, jnp.float32)
                        new[2 * c] = new[2 * c] + ev * wv
                        new[2 * c + 1] = new[2 * c + 1] + od * wv
                    accs = tuple(new)
                return accs

            def cbody(c, accs):
                dchunk = dbuf[pl.ds(c * 16, 16)].astype(jnp.float32)
                w = jnp.exp(-jnp.abs(dchunk - CURRENT_DAY) * rate)
                return accum_lanes(w, c * 16, accs, range(16))

            zeros = tuple(jnp.zeros((16,), jnp.float32) for _ in range(NACC))
            accs = plsc.parallel_loop(0, (L - 8) // 16, unroll=2,
                                      carry=zeros)(cbody)

            dtail = dbuf[pl.ds(L - 16, 16)].astype(jnp.float32)
            wtail = jnp.exp(-jnp.abs(dtail - CURRENT_DAY) * rate)
            accs = accum_lanes(wtail, L - 16, accs, range(8, 16))

            stot = jnp.zeros((16,), jnp.float32)
            for c in range(NACC // 2):
                ev = accs[2 * c]
                od = accs[2 * c + 1]
                u_a = u_rows[b, pl.ds(c * 32, 16)]
                u_b = u_rows[b, pl.ds(c * 32 + 16, 16)]
                u_ev = jnp.where(lane < 8,
                                 jnp.take(u_a, idx_even,
                                          mode="fill"),
                                 jnp.take(u_b, idx_even,
                                          mode="fill"))
                u_od = jnp.where(lane < 8,
                                 jnp.take(u_a, idx_odd,
                                          mode="fill"),
                                 jnp.take(u_b, idx_odd,
                                          mode="fill"))
                stot = stot + ev * u_ev + od * u_od
            sp = vsum(stot) * (1.0 / L)
            return jnp.clip(sp.astype(jnp.int32), 0, SESSIONS - 1)

        start_gather(0, e0, d0, sem0)

        def gbody(g, sidx_acc):
            b0 = 2 * g
            b1 = b0 + 1
            start_gather(b1, e1, d1, sem1)
            wait_gather(b0, e0, d0, sem0)
            i0 = compute_b(b0, e0, d0)
            sidx_acc = jnp.where(lane == b0 % 16, i0, sidx_acc)

            @pl.when(b0 + 2 < per_w)
            def _():
                start_gather(b0 + 2, e0, d0, sem0)

            wait_gather(b1, e1, d1, sem1)
            i1 = compute_b(b1, e1, d1)
            sidx_acc = jnp.where(lane == b1 % 16, i1, sidx_acc)

            @pl.when(b1 % 16 == 15)
            def _():
                sidx_v[pl.ds(b1 - 15, 16)] = sidx_acc

            return sidx_acc

        lax.fori_loop(0, per_w // 2, gbody, jnp.zeros((16,), jnp.int32))

        cp_sb = pltpu.make_async_copy(sbias_hbm.at[sidx_v], sb_v, semm)
        cp_ie = pltpu.make_async_copy(iemb_hbm.at[iid_v], i_rows, semm)
        cp_sb.start()
        cp_ie.start()
        cp_sb.wait()
        cp_ie.wait()

        def rbody(g, carry):
            raws = jnp.zeros((16,), jnp.float32)
            for k in range(16):
                r = row_dot(u_rows, 16 * g + k,
                            tuple(i_rows[16 * g + k, pl.ds(c * 16, 16)]
                                  for c in range(NACC)))
                raws = jnp.where(lane == k, r, raws)
            off = pl.ds(16 * g, 16)
            out_v[off] = (raws + AVG_RATING
                          + ub_v[off] + ib_v[off] + sb_v[off])
            return carry

        lax.fori_loop(0, per_w // 16, rbody, 0)
        pltpu.sync_copy(out_v, out_hbm.at[pl.ds(base, per_w)])

    return sc_call


def kernel(user_id, item_id, session_items_ids, session_items_daystamps,
           user_emb_table, item_emb_table, user_bias_table, item_bias_table,
           session_bias_table, user_decay_table):
    info = plsc.get_sparse_core_info()
    sid3 = session_items_ids.reshape(B, NCHUNK, LJ)
    sc_call = _build_sc_call(info.num_cores, info.num_subcores)
    item_pk = lax.bitcast_convert_type(
        item_emb_table.astype(jnp.bfloat16).reshape(ITEMS, D // 2, 2),
        jnp.int32)
    return sc_call(user_id, item_id, sid3, session_items_daystamps,
                   user_emb_table, item_emb_table, item_pk,
                   user_bias_table.reshape(USERS),
                   item_bias_table.reshape(ITEMS),
                   session_bias_table.reshape(SESSIONS),
                   user_decay_table.reshape(USERS))


# speedup vs baseline: 1.3357x; 1.3357x over previous
import functools

import jax
import jax.numpy as jnp
from jax import lax
from jax.experimental import pallas as pl
from jax.experimental.pallas import tpu as pltpu
from jax.experimental.pallas import tpu_sc as plsc

USERS = 100000
ITEMS = 100000
SESSIONS = 100000
D = 128
B = 4096
L = 200
AVG_RATING = 3.5
CURRENT_DAY = 17990.0

LJ = 100
NCHUNK = 2
CHUNK_OFF = (0, 100)
NACC = D // 16


def _build_sc_call(nc, ns):
    nw = nc * ns
    per_w = B // nw
    mesh = plsc.VectorSubcoreMesh(core_axis_name="c", subcore_axis_name="s")

    @functools.partial(
        pl.kernel,
        mesh=mesh,
        out_type=jax.ShapeDtypeStruct((B,), jnp.float32),
        compiler_params=pltpu.CompilerParams(needs_layout_passes=False,
                                             use_tc_tiling_on_sc=False),
        scratch_types=[
            pltpu.VMEM((per_w, NCHUNK, LJ), jnp.int32),
            pltpu.VMEM((L,), jnp.int32),
            pltpu.VMEM((L,), jnp.int32),
            pltpu.VMEM((per_w, D), jnp.float32),
            pltpu.VMEM((per_w, D), jnp.float32),
            pltpu.VMEM((L, D // 2), jnp.int32),
            pltpu.VMEM((L, D // 2), jnp.int32),
            pltpu.VMEM((per_w,), jnp.int32),
            pltpu.VMEM((per_w,), jnp.int32),
            pltpu.VMEM((per_w,), jnp.int32),
            pltpu.VMEM((per_w,), jnp.float32),
            pltpu.VMEM((per_w,), jnp.float32),
            pltpu.VMEM((per_w,), jnp.float32),
            pltpu.VMEM((per_w,), jnp.float32),
            pltpu.VMEM((per_w,), jnp.float32),
            pltpu.SemaphoreType.DMA,
            pltpu.SemaphoreType.DMA,
            pltpu.SemaphoreType.DMA,
        ],
    )
    def sc_call(uid_hbm, iid_hbm, sid_hbm, days_hbm, uemb_hbm, iemb_hbm,
                iembbf_hbm, ubias_hbm, ibias_hbm, sbias_hbm, udec_hbm,
                out_hbm,
                ids_v, d0, d1, u_rows, i_rows, e0, e1, uid_v, iid_v, sidx_v,
                dec_v, ub_v, ib_v, sb_v, out_v, sem0, sem1, semm):
        wid = lax.axis_index("s") * nc + lax.axis_index("c")
        base = wid * per_w
        lane = lax.iota(jnp.int32, 16)
        idx_even = (2 * lane) & 15
        idx_odd = (2 * lane + 1) & 15
        himask = jnp.full((16,), jnp.int32(-65536))
        splats = [jnp.full((16,), jnp.int32(k)) for k in range(16)]

        pltpu.sync_copy(uid_hbm.at[pl.ds(base, per_w)], uid_v)
        pltpu.sync_copy(iid_hbm.at[pl.ds(base, per_w)], iid_v)
        pltpu.sync_copy(sid_hbm.at[pl.ds(base, per_w)], ids_v)

        cp_u = pltpu.make_async_copy(uemb_hbm.at[uid_v], u_rows, semm)
        cp_d = pltpu.make_async_copy(udec_hbm.at[uid_v], dec_v, semm)
        cp_ub = pltpu.make_async_copy(ubias_hbm.at[uid_v], ub_v, semm)
        cp_ib = pltpu.make_async_copy(ibias_hbm.at[iid_v], ib_v, semm)
        for cp in (cp_u, cp_d, cp_ub, cp_ib):
            cp.start()
        for cp in (cp_u, cp_d, cp_ub, cp_ib):
            cp.wait()

        def start_gather(b, ebuf, dbuf, sem):
            for j in range(NCHUNK):
                pltpu.make_async_copy(
                    iembbf_hbm.at[ids_v.at[b, j]],
                    ebuf.at[pl.ds(CHUNK_OFF[j], LJ)], sem).start()
            pltpu.make_async_copy(days_hbm.at[base + b], dbuf, sem).start()

        def wait_gather(b, ebuf, dbuf, sem):
            for j in range(NCHUNK):
                pltpu.make_async_copy(
                    iembbf_hbm.at[ids_v.at[b, j]],
                    ebuf.at[pl.ds(CHUNK_OFF[j], LJ)], sem).wait()
            pltpu.make_async_copy(days_hbm.at[base + b], dbuf, sem).wait()

        def vsum(v):
            return plsc.cumsum(v)[15]

        def row_dot(row_a, b, accs):
            s = jnp.zeros((16,), jnp.float32)
            for c in range(NACC):
                s = s + accs[c] * row_a[b, pl.ds(c * 16, 16)]
            return vsum(s)

        def compute_b(b, ebuf, dbuf):
            grp = (b // 16) * 16
            rchunk = dec_v[pl.ds(grp, 16)]
            rate = vsum(jnp.where(lane == b - grp, rchunk, 0.0))
            rate = jnp.maximum(rate, 0.0)

            def accum_lanes(w, l0, accs, lanes):
                for k in lanes:
                    wv = jnp.take(w, splats[k], mode="fill")
                    wb = plsc.pack(wv, wv, format=plsc.PackFormat.INTERLEAVED)
                    l = l0 + k
                    new = list(accs)
                    for c in range(NACC // 2):
                        chunk = plsc.bitcast(ebuf[l, pl.ds(c * 16, 16)],
                                             jnp.bfloat16)
                        new[c] = new[c] + chunk * wb
                    accs = tuple(new)
                return accs

            def cbody(c, accs):
                dchunk = dbuf[pl.ds(c * 16, 16)].astype(jnp.float32)
                w = jnp.exp(-jnp.abs(dchunk - CURRENT_DAY) * rate)
                return accum_lanes(w, c * 16, accs, range(16))

            zeros = tuple(jnp.zeros((32,), jnp.bfloat16)
                          for _ in range(NACC // 2))
            accs = plsc.parallel_loop(0, (L - 8) // 16, unroll=2,
                                      carry=zeros)(cbody)

            dtail = dbuf[pl.ds(L - 16, 16)].astype(jnp.float32)
            wtail = jnp.exp(-jnp.abs(dtail - CURRENT_DAY) * rate)
            accs = accum_lanes(wtail, L - 16, accs, range(8, 16))

            stot = jnp.zeros((16,), jnp.float32)
            for c in range(NACC // 2):
                ev, od = plsc.unpack(accs[c],
                                     format=plsc.PackFormat.INTERLEAVED)
                u_a = u_rows[b, pl.ds(c * 32, 16)]
                u_b = u_rows[b, pl.ds(c * 32 + 16, 16)]
                u_ev = jnp.where(lane < 8,
                                 jnp.take(u_a, idx_even,
                                          mode="fill"),
                                 jnp.take(u_b, idx_even,
                                          mode="fill"))
                u_od = jnp.where(lane < 8,
                                 jnp.take(u_a, idx_odd,
                                          mode="fill"),
                                 jnp.take(u_b, idx_odd,
                                          mode="fill"))
                stot = stot + ev * u_ev + od * u_od
            sp = vsum(stot) * (1.0 / L)
            return jnp.clip(sp.astype(jnp.int32), 0, SESSIONS - 1)

        start_gather(0, e0, d0, sem0)

        def gbody(g, sidx_acc):
            b0 = 2 * g
            b1 = b0 + 1
            start_gather(b1, e1, d1, sem1)
            wait_gather(b0, e0, d0, sem0)
            i0 = compute_b(b0, e0, d0)
            sidx_acc = jnp.where(lane == b0 % 16, i0, sidx_acc)

            @pl.when(b0 + 2 < per_w)
            def _():
                start_gather(b0 + 2, e0, d0, sem0)

            wait_gather(b1, e1, d1, sem1)
            i1 = compute_b(b1, e1, d1)
            sidx_acc = jnp.where(lane == b1 % 16, i1, sidx_acc)

            @pl.when(b1 % 16 == 15)
            def _():
                sidx_v[pl.ds(b1 - 15, 16)] = sidx_acc

            return sidx_acc

        lax.fori_loop(0, per_w // 2, gbody, jnp.zeros((16,), jnp.int32))

        cp_sb = pltpu.make_async_copy(sbias_hbm.at[sidx_v], sb_v, semm)
        cp_ie = pltpu.make_async_copy(iemb_hbm.at[iid_v], i_rows, semm)
        cp_sb.start()
        cp_ie.start()
        cp_sb.wait()
        cp_ie.wait()

        def rbody(g, carry):
            raws = jnp.zeros((16,), jnp.float32)
            for k in range(16):
                r = row_dot(u_rows, 16 * g + k,
                            tuple(i_rows[16 * g + k, pl.ds(c * 16, 16)]
                                  for c in range(NACC)))
                raws = jnp.where(lane == k, r, raws)
            off = pl.ds(16 * g, 16)
            out_v[off] = (raws + AVG_RATING
                          + ub_v[off] + ib_v[off] + sb_v[off])
            return carry

        lax.fori_loop(0, per_w // 16, rbody, 0)
        pltpu.sync_copy(out_v, out_hbm.at[pl.ds(base, per_w)])

    return sc_call


def kernel(user_id, item_id, session_items_ids, session_items_daystamps,
           user_emb_table, item_emb_table, user_bias_table, item_bias_table,
           session_bias_table, user_decay_table):
    info = plsc.get_sparse_core_info()
    sid3 = session_items_ids.reshape(B, NCHUNK, LJ)
    sc_call = _build_sc_call(info.num_cores, info.num_subcores)
    item_pk = lax.bitcast_convert_type(
        item_emb_table.astype(jnp.bfloat16).reshape(ITEMS, D // 2, 2),
        jnp.int32)
    return sc_call(user_id, item_id, sid3, session_items_daystamps,
                   user_emb_table, item_emb_table, item_pk,
                   user_bias_table.reshape(USERS),
                   item_bias_table.reshape(ITEMS),
                   session_bias_table.reshape(SESSIONS),
                   user_decay_table.reshape(USERS))


# speedup vs baseline: 1.3439x; 1.0061x over previous
import functools

import jax
import jax.numpy as jnp
from jax import lax
from jax.experimental import pallas as pl
from jax.experimental.pallas import tpu as pltpu
from jax.experimental.pallas import tpu_sc as plsc

USERS = 100000
ITEMS = 100000
SESSIONS = 100000
D = 128
B = 4096
L = 200
AVG_RATING = 3.5
CURRENT_DAY = 17990.0

LJ = 100
NCHUNK = 2
CHUNK_OFF = (0, 100)
NACC = D // 16


def _build_sc_call(nc, ns):
    nw = nc * ns
    per_w = B // nw
    mesh = plsc.VectorSubcoreMesh(core_axis_name="c", subcore_axis_name="s")

    @functools.partial(
        pl.kernel,
        mesh=mesh,
        out_type=jax.ShapeDtypeStruct((B,), jnp.float32),
        compiler_params=pltpu.CompilerParams(needs_layout_passes=False,
                                             use_tc_tiling_on_sc=False),
        scratch_types=[
            pltpu.VMEM((per_w, NCHUNK, LJ), jnp.int32),
            pltpu.VMEM((L,), jnp.int32),
            pltpu.VMEM((L,), jnp.int32),
            pltpu.VMEM((per_w, D), jnp.float32),
            pltpu.VMEM((per_w, D), jnp.float32),
            pltpu.VMEM((L, D // 2), jnp.int32),
            pltpu.VMEM((L, D // 2), jnp.int32),
            pltpu.VMEM((per_w,), jnp.int32),
            pltpu.VMEM((per_w,), jnp.int32),
            pltpu.VMEM((per_w,), jnp.int32),
            pltpu.VMEM((per_w,), jnp.float32),
            pltpu.VMEM((per_w,), jnp.float32),
            pltpu.VMEM((per_w,), jnp.float32),
            pltpu.VMEM((per_w,), jnp.float32),
            pltpu.VMEM((per_w,), jnp.float32),
            pltpu.SemaphoreType.DMA,
            pltpu.SemaphoreType.DMA,
            pltpu.SemaphoreType.DMA,
        ],
    )
    def sc_call(uid_hbm, iid_hbm, sid_hbm, days_hbm, uemb_hbm, iemb_hbm,
                iembbf_hbm, ubias_hbm, ibias_hbm, sbias_hbm, udec_hbm,
                out_hbm,
                ids_v, d0, d1, u_rows, i_rows, e0, e1, uid_v, iid_v, sidx_v,
                dec_v, ub_v, ib_v, sb_v, out_v, sem0, sem1, semm):
        wid = lax.axis_index("s") * nc + lax.axis_index("c")
        base = wid * per_w
        lane = lax.iota(jnp.int32, 16)
        idx_even = (2 * lane) & 15
        idx_odd = (2 * lane + 1) & 15
        himask = jnp.full((16,), jnp.int32(-65536))
        splats = [jnp.full((16,), jnp.int32(k)) for k in range(16)]

        pltpu.sync_copy(uid_hbm.at[pl.ds(base, per_w)], uid_v)
        pltpu.sync_copy(iid_hbm.at[pl.ds(base, per_w)], iid_v)
        pltpu.sync_copy(sid_hbm.at[pl.ds(base, per_w)], ids_v)

        cp_u = pltpu.make_async_copy(uemb_hbm.at[uid_v], u_rows, semm)
        cp_d = pltpu.make_async_copy(udec_hbm.at[uid_v], dec_v, semm)
        cp_ub = pltpu.make_async_copy(ubias_hbm.at[uid_v], ub_v, semm)
        cp_ib = pltpu.make_async_copy(ibias_hbm.at[iid_v], ib_v, semm)
        for cp in (cp_u, cp_d, cp_ub, cp_ib):
            cp.start()
        for cp in (cp_u, cp_d, cp_ub, cp_ib):
            cp.wait()

        def start_gather(b, ebuf, dbuf, sem):
            for j in range(NCHUNK):
                pltpu.make_async_copy(
                    iembbf_hbm.at[ids_v.at[b, j]],
                    ebuf.at[pl.ds(CHUNK_OFF[j], LJ)], sem).start()
            pltpu.make_async_copy(days_hbm.at[base + b], dbuf, sem).start()

        def wait_gather(b, ebuf, dbuf, sem):
            for j in range(NCHUNK):
                pltpu.make_async_copy(
                    iembbf_hbm.at[ids_v.at[b, j]],
                    ebuf.at[pl.ds(CHUNK_OFF[j], LJ)], sem).wait()
            pltpu.make_async_copy(days_hbm.at[base + b], dbuf, sem).wait()

        def vsum(v):
            return plsc.cumsum(v)[15]

        def row_dot(row_a, b, accs):
            s = jnp.zeros((16,), jnp.float32)
            for c in range(NACC):
                s = s + accs[c] * row_a[b, pl.ds(c * 16, 16)]
            return vsum(s)

        def compute_b(b, ebuf, dbuf):
            grp = (b // 16) * 16
            rchunk = dec_v[pl.ds(grp, 16)]
            rate = vsum(jnp.where(lane == b - grp, rchunk, 0.0))
            rate = jnp.maximum(rate, 0.0)

            def accum_lanes(w, l0, accs, lanes):
                wpk = plsc.bitcast(
                    plsc.pack(w, w, format=plsc.PackFormat.INTERLEAVED),
                    jnp.int32)
                for k in lanes:
                    wb = plsc.bitcast(jnp.take(wpk, splats[k], mode="fill"),
                                      jnp.bfloat16)
                    l = l0 + k
                    new = list(accs)
                    for c in range(NACC // 2):
                        chunk = plsc.bitcast(ebuf[l, pl.ds(c * 16, 16)],
                                             jnp.bfloat16)
                        new[c] = new[c] + chunk * wb
                    accs = tuple(new)
                return accs

            def cbody(c, accs):
                dchunk = dbuf[pl.ds(c * 16, 16)].astype(jnp.float32)
                w = jnp.exp(-jnp.abs(dchunk - CURRENT_DAY) * rate)
                return accum_lanes(w, c * 16, accs, range(16))

            zeros = tuple(jnp.zeros((32,), jnp.bfloat16)
                          for _ in range(NACC // 2))
            accs = plsc.parallel_loop(0, (L - 8) // 16, unroll=2,
                                      carry=zeros)(cbody)

            dtail = dbuf[pl.ds(L - 16, 16)].astype(jnp.float32)
            wtail = jnp.exp(-jnp.abs(dtail - CURRENT_DAY) * rate)
            accs = accum_lanes(wtail, L - 16, accs, range(8, 16))

            stot = jnp.zeros((16,), jnp.float32)
            for c in range(NACC // 2):
                ev, od = plsc.unpack(accs[c],
                                     format=plsc.PackFormat.INTERLEAVED)
                u_a = u_rows[b, pl.ds(c * 32, 16)]
                u_b = u_rows[b, pl.ds(c * 32 + 16, 16)]
                u_ev = jnp.where(lane < 8,
                                 jnp.take(u_a, idx_even,
                                          mode="fill"),
                                 jnp.take(u_b, idx_even,
                                          mode="fill"))
                u_od = jnp.where(lane < 8,
                                 jnp.take(u_a, idx_odd,
                                          mode="fill"),
                                 jnp.take(u_b, idx_odd,
                                          mode="fill"))
                stot = stot + ev * u_ev + od * u_od
            sp = vsum(stot) * (1.0 / L)
            return jnp.clip(sp.astype(jnp.int32), 0, SESSIONS - 1)

        start_gather(0, e0, d0, sem0)

        def gbody(g, sidx_acc):
            b0 = 2 * g
            b1 = b0 + 1
            start_gather(b1, e1, d1, sem1)
            wait_gather(b0, e0, d0, sem0)
            i0 = compute_b(b0, e0, d0)
            sidx_acc = jnp.where(lane == b0 % 16, i0, sidx_acc)

            @pl.when(b0 + 2 < per_w)
            def _():
                start_gather(b0 + 2, e0, d0, sem0)

            wait_gather(b1, e1, d1, sem1)
            i1 = compute_b(b1, e1, d1)
            sidx_acc = jnp.where(lane == b1 % 16, i1, sidx_acc)

            @pl.when(b1 % 16 == 15)
            def _():
                sidx_v[pl.ds(b1 - 15, 16)] = sidx_acc

            return sidx_acc

        lax.fori_loop(0, per_w // 2, gbody, jnp.zeros((16,), jnp.int32))

        cp_sb = pltpu.make_async_copy(sbias_hbm.at[sidx_v], sb_v, semm)
        cp_ie = pltpu.make_async_copy(iemb_hbm.at[iid_v], i_rows, semm)
        cp_sb.start()
        cp_ie.start()
        cp_sb.wait()
        cp_ie.wait()

        def rbody(g, carry):
            raws = jnp.zeros((16,), jnp.float32)
            for k in range(16):
                r = row_dot(u_rows, 16 * g + k,
                            tuple(i_rows[16 * g + k, pl.ds(c * 16, 16)]
                                  for c in range(NACC)))
                raws = jnp.where(lane == k, r, raws)
            off = pl.ds(16 * g, 16)
            out_v[off] = (raws + AVG_RATING
                          + ub_v[off] + ib_v[off] + sb_v[off])
            return carry

        lax.fori_loop(0, per_w // 16, rbody, 0)
        pltpu.sync_copy(out_v, out_hbm.at[pl.ds(base, per_w)])

    return sc_call


def kernel(user_id, item_id, session_items_ids, session_items_daystamps,
           user_emb_table, item_emb_table, user_bias_table, item_bias_table,
           session_bias_table, user_decay_table):
    info = plsc.get_sparse_core_info()
    sid3 = session_items_ids.reshape(B, NCHUNK, LJ)
    sc_call = _build_sc_call(info.num_cores, info.num_subcores)
    item_pk = lax.bitcast_convert_type(
        item_emb_table.astype(jnp.bfloat16).reshape(ITEMS, D // 2, 2),
        jnp.int32)
    return sc_call(user_id, item_id, sid3, session_items_daystamps,
                   user_emb_table, item_emb_table, item_pk,
                   user_bias_table.reshape(USERS),
                   item_bias_table.reshape(ITEMS),
                   session_bias_table.reshape(SESSIONS),
                   user_decay_table.reshape(USERS))


# speedup vs baseline: 2.4934x; 1.8554x over previous
import functools

import jax
import jax.numpy as jnp
from jax import lax
from jax.experimental import pallas as pl
from jax.experimental.pallas import tpu as pltpu
from jax.experimental.pallas import tpu_sc as plsc

USERS = 100000
ITEMS = 100000
SESSIONS = 100000
D = 128
B = 4096
L = 200
AVG_RATING = 3.5
CURRENT_DAY = 17990.0

LJ = 100
NCHUNK = L // LJ
NACC = D // 16


def _build_sc_call(nc, ns):
    nw = nc * ns
    per_w = B // nw
    mesh = plsc.VectorSubcoreMesh(core_axis_name="c", subcore_axis_name="s")

    @functools.partial(
        pl.kernel,
        mesh=mesh,
        out_type=jax.ShapeDtypeStruct((B,), jnp.float32),
        compiler_params=pltpu.CompilerParams(needs_layout_passes=False),
        scratch_types=[
            pltpu.VMEM((per_w, NCHUNK, LJ), jnp.int32),
            pltpu.VMEM((L,), jnp.int32),
            pltpu.VMEM((L,), jnp.int32),
            pltpu.VMEM((per_w, D), jnp.float32),
            pltpu.VMEM((L, D), jnp.float32),
            pltpu.VMEM((L, D), jnp.float32),
            pltpu.VMEM((per_w,), jnp.int32),
            pltpu.VMEM((per_w,), jnp.int32),
            pltpu.VMEM((per_w,), jnp.int32),
            pltpu.VMEM((per_w,), jnp.float32),
            pltpu.VMEM((per_w,), jnp.float32),
            pltpu.VMEM((per_w,), jnp.float32),
            pltpu.VMEM((per_w,), jnp.float32),
            pltpu.VMEM((per_w,), jnp.float32),
            pltpu.SemaphoreType.DMA,
            pltpu.SemaphoreType.DMA,
            pltpu.SemaphoreType.DMA,
        ],
    )
    def sc_call(uid_hbm, iid_hbm, sid_hbm, days_hbm, uemb_hbm, iemb_hbm,
                ubias_hbm, ibias_hbm, sbias_hbm, udec_hbm, out_hbm,
                ids_v, d0, d1, u_rows, e0, e1, uid_v, iid_v, sidx_v,
                dec_v, ub_v, ib_v, sb_v, out_v, sem0, sem1, semm):
        wid = lax.axis_index("s") * nc + lax.axis_index("c")
        base = wid * per_w
        lane = lax.iota(jnp.int32, 16)
        splats = [jnp.full((16,), jnp.int32(k)) for k in range(16)]

        pltpu.sync_copy(uid_hbm.at[pl.ds(base, per_w)], uid_v)
        pltpu.sync_copy(iid_hbm.at[pl.ds(base, per_w)], iid_v)
        pltpu.sync_copy(sid_hbm.at[pl.ds(base, per_w)], ids_v)

        cp_u = pltpu.make_async_copy(uemb_hbm.at[uid_v], u_rows, semm)
        cp_d = pltpu.make_async_copy(udec_hbm.at[uid_v], dec_v, semm)
        cp_ub = pltpu.make_async_copy(ubias_hbm.at[uid_v], ub_v, semm)
        cp_ib = pltpu.make_async_copy(ibias_hbm.at[iid_v], ib_v, semm)
        for cp in (cp_u, cp_d, cp_ub, cp_ib):
            cp.start()
        for cp in (cp_u, cp_d, cp_ub, cp_ib):
            cp.wait()

        def start_gather(b, ebuf, dbuf, sem):
            for j in range(NCHUNK):
                pltpu.make_async_copy(
                    iemb_hbm.at[ids_v.at[b, j]],
                    ebuf.at[pl.ds(j * LJ, LJ)], sem).start()
            pltpu.make_async_copy(days_hbm.at[base + b], dbuf, sem).start()

        def wait_gather(b, ebuf, dbuf, sem):
            for j in range(NCHUNK):
                pltpu.make_async_copy(
                    iemb_hbm.at[ids_v.at[b, j]],
                    ebuf.at[pl.ds(j * LJ, LJ)], sem).wait()
            pltpu.make_async_copy(days_hbm.at[base + b], dbuf, sem).wait()

        def vsum(v):
            return plsc.cumsum(v)[15]

        def row_dot(row_a, b, accs):
            s = jnp.zeros((16,), jnp.float32)
            for c in range(NACC):
                s = s + accs[c] * row_a[b, pl.ds(c * 16, 16)]
            return vsum(s)

        def compute_b(b, ebuf, dbuf):
            grp = (b // 16) * 16
            rchunk = dec_v[pl.ds(grp, 16)]
            rate = vsum(jnp.where(lane == b - grp, rchunk, 0.0))
            rate = jnp.maximum(rate, 0.0)

            def accum_lanes(w, l0, accs, lanes):
                for k in lanes:
                    wv = jnp.take(w, splats[k], mode="fill")
                    l = l0 + k
                    accs = tuple(accs[c] + ebuf[l, pl.ds(c * 16, 16)] * wv
                                 for c in range(NACC))
                return accs

            def cbody(c, accs):
                dchunk = dbuf[pl.ds(c * 16, 16)].astype(jnp.float32)
                w = jnp.exp(-jnp.abs(dchunk - CURRENT_DAY) * rate)
                return accum_lanes(w, c * 16, accs, range(16))

            zeros = tuple(jnp.zeros((16,), jnp.float32) for _ in range(NACC))
            accs = plsc.parallel_loop(0, (L - 8) // 16, unroll=4,
                                      carry=zeros)(cbody)

            dtail = dbuf[pl.ds(L - 16, 16)].astype(jnp.float32)
            wtail = jnp.exp(-jnp.abs(dtail - CURRENT_DAY) * rate)
            accs = accum_lanes(wtail, L - 16, accs, range(8, 16))

            sp = row_dot(u_rows, b, accs) * (1.0 / L)
            return jnp.clip(sp.astype(jnp.int32), 0, SESSIONS - 1)

        start_gather(0, e0, d0, sem0)

        def gbody(g, sidx_acc):
            b0 = 2 * g
            b1 = b0 + 1
            start_gather(b1, e1, d1, sem1)
            wait_gather(b0, e0, d0, sem0)
            i0 = compute_b(b0, e0, d0)
            sidx_acc = jnp.where(lane == b0 % 16, i0, sidx_acc)

            @pl.when(b0 + 2 < per_w)
            def _():
                start_gather(b0 + 2, e0, d0, sem0)

            wait_gather(b1, e1, d1, sem1)
            i1 = compute_b(b1, e1, d1)
            sidx_acc = jnp.where(lane == b1 % 16, i1, sidx_acc)

            @pl.when(b1 % 16 == 15)
            def _():
                sidx_v[pl.ds(b1 - 15, 16)] = sidx_acc

            return sidx_acc

        lax.fori_loop(0, per_w // 2, gbody, jnp.zeros((16,), jnp.int32))

        cp_sb = pltpu.make_async_copy(sbias_hbm.at[sidx_v], sb_v, semm)
        cp_ie = pltpu.make_async_copy(iemb_hbm.at[iid_v],
                                      e0.at[pl.ds(0, per_w)], semm)
        cp_sb.start()
        cp_ie.start()
        cp_sb.wait()
        cp_ie.wait()

        def rbody(g, carry):
            raws = jnp.zeros((16,), jnp.float32)
            for k in range(16):
                r = row_dot(u_rows, 16 * g + k,
                            tuple(e0[16 * g + k, pl.ds(c * 16, 16)]
                                  for c in range(NACC)))
                raws = jnp.where(lane == k, r, raws)
            off = pl.ds(16 * g, 16)
            out_v[off] = (raws + AVG_RATING
                          + ub_v[off] + ib_v[off] + sb_v[off])
            return carry

        lax.fori_loop(0, per_w // 16, rbody, 0)
        pltpu.sync_copy(out_v, out_hbm.at[pl.ds(base, per_w)])

    return sc_call


def kernel(user_id, item_id, session_items_ids, session_items_daystamps,
           user_emb_table, item_emb_table, user_bias_table, item_bias_table,
           session_bias_table, user_decay_table):
    info = plsc.get_sparse_core_info()
    sid3 = session_items_ids.reshape(B, NCHUNK, LJ)
    sc_call = _build_sc_call(info.num_cores, info.num_subcores)
    return sc_call(user_id, item_id, sid3, session_items_daystamps,
                   user_emb_table, item_emb_table,
                   user_bias_table.reshape(USERS),
                   item_bias_table.reshape(ITEMS),
                   session_bias_table.reshape(SESSIONS),
                   user_decay_table.reshape(USERS))


# speedup vs baseline: 3.5680x; 1.4310x over previous
import functools

import jax
import jax.numpy as jnp
from jax import lax
from jax.experimental import pallas as pl
from jax.experimental.pallas import tpu as pltpu
from jax.experimental.pallas import tpu_sc as plsc

USERS = 100000
ITEMS = 100000
SESSIONS = 100000
D = 128
B = 4096
L = 200
AVG_RATING = 3.5
CURRENT_DAY = 17990.0

LJ = 100
NCHUNK = L // LJ
NACC = D // 16


def _build_sc_call(nc, ns):
    nw = nc * ns
    per_w = B // nw
    mesh = plsc.VectorSubcoreMesh(core_axis_name="c", subcore_axis_name="s")

    @functools.partial(
        pl.kernel,
        mesh=mesh,
        out_type=jax.ShapeDtypeStruct((B,), jnp.float32),
        compiler_params=pltpu.CompilerParams(needs_layout_passes=False),
        scratch_types=[
            pltpu.VMEM((per_w, NCHUNK, LJ), jnp.int32),
            pltpu.VMEM((L,), jnp.int32),
            pltpu.VMEM((L,), jnp.int32),
            pltpu.VMEM((per_w, D), jnp.float32),
            pltpu.VMEM((L, D), jnp.float32),
            pltpu.VMEM((L, D), jnp.float32),
            pltpu.VMEM((per_w,), jnp.int32),
            pltpu.VMEM((per_w,), jnp.int32),
            pltpu.VMEM((per_w,), jnp.int32),
            pltpu.VMEM((per_w,), jnp.float32),
            pltpu.VMEM((per_w,), jnp.float32),
            pltpu.VMEM((per_w,), jnp.float32),
            pltpu.VMEM((per_w,), jnp.float32),
            pltpu.VMEM((per_w,), jnp.float32),
            pltpu.SemaphoreType.DMA,
            pltpu.SemaphoreType.DMA,
            pltpu.SemaphoreType.DMA,
        ],
    )
    def sc_call(uid_hbm, iid_hbm, sid_hbm, days_hbm, uemb_hbm, iemb_hbm,
                ubias_hbm, ibias_hbm, sbias_hbm, udec_hbm, out_hbm,
                ids_v, d0, d1, u_rows, e0, e1, uid_v, iid_v, sidx_v,
                dec_v, ub_v, ib_v, sb_v, out_v, sem0, sem1, semm):
        wid = lax.axis_index("s") * nc + lax.axis_index("c")
        base = wid * per_w
        lane = lax.iota(jnp.int32, 16)

        pltpu.sync_copy(uid_hbm.at[pl.ds(base, per_w)], uid_v)
        pltpu.sync_copy(iid_hbm.at[pl.ds(base, per_w)], iid_v)
        pltpu.sync_copy(sid_hbm.at[pl.ds(base, per_w)], ids_v)

        cp_u = pltpu.make_async_copy(uemb_hbm.at[uid_v], u_rows, semm)
        cp_d = pltpu.make_async_copy(udec_hbm.at[uid_v], dec_v, semm)
        cp_ub = pltpu.make_async_copy(ubias_hbm.at[uid_v], ub_v, semm)
        cp_ib = pltpu.make_async_copy(ibias_hbm.at[iid_v], ib_v, semm)
        for cp in (cp_u, cp_d, cp_ub, cp_ib):
            cp.start()
        for cp in (cp_u, cp_d, cp_ub, cp_ib):
            cp.wait()

        def start_gather(b, ebuf, dbuf, sem):
            for j in range(NCHUNK):
                pltpu.make_async_copy(
                    iemb_hbm.at[ids_v.at[b, j]],
                    ebuf.at[pl.ds(j * LJ, LJ)], sem).start()
            pltpu.make_async_copy(days_hbm.at[base + b], dbuf, sem).start()

        def wait_gather(b, ebuf, dbuf, sem):
            for j in range(NCHUNK):
                pltpu.make_async_copy(
                    iemb_hbm.at[ids_v.at[b, j]],
                    ebuf.at[pl.ds(j * LJ, LJ)], sem).wait()
            pltpu.make_async_copy(days_hbm.at[base + b], dbuf, sem).wait()

        def vsum(v):
            return plsc.cumsum(v)[15]

        def row_dot(row_a, b, accs):
            s = jnp.zeros((16,), jnp.float32)
            for c in range(NACC):
                s = s + accs[c] * row_a[b, pl.ds(c * 16, 16)]
            return vsum(s)

        def compute_b(b, ebuf, dbuf):
            grp = (b // 16) * 16
            rchunk = dec_v[pl.ds(grp, 16)]
            rate = vsum(jnp.where(lane == b - grp, rchunk, 0.0))
            rate = jnp.maximum(rate, 0.0)

            def accum_lanes(w, l0, accs, lanes):
                for k in lanes:
                    wl = w[k]
                    l = l0 + k
                    accs = tuple(accs[c] + ebuf[l, pl.ds(c * 16, 16)] * wl
                                 for c in range(NACC))
                return accs

            def cbody(c, accs):
                dchunk = dbuf[pl.ds(c * 16, 16)].astype(jnp.float32)
                w = jnp.exp(-jnp.abs(dchunk - CURRENT_DAY) * rate)
                return accum_lanes(w, c * 16, accs, range(16))

            zeros = tuple(jnp.zeros((16,), jnp.float32) for _ in range(NACC))
            accs = lax.fori_loop(0, (L - 8) // 16, cbody, zeros)

            dtail = dbuf[pl.ds(L - 16, 16)].astype(jnp.float32)
            wtail = jnp.exp(-jnp.abs(dtail - CURRENT_DAY) * rate)
            accs = accum_lanes(wtail, L - 16, accs, range(8, 16))

            sp = row_dot(u_rows, b, accs) * (1.0 / L)
            return jnp.clip(sp.astype(jnp.int32), 0, SESSIONS - 1)

        start_gather(0, e0, d0, sem0)

        def gbody(g, sidx_acc):
            b0 = 2 * g
            b1 = b0 + 1
            start_gather(b1, e1, d1, sem1)
            wait_gather(b0, e0, d0, sem0)
            i0 = compute_b(b0, e0, d0)
            sidx_acc = jnp.where(lane == b0 % 16, i0, sidx_acc)

            @pl.when(b0 + 2 < per_w)
            def _():
                start_gather(b0 + 2, e0, d0, sem0)

            wait_gather(b1, e1, d1, sem1)
            i1 = compute_b(b1, e1, d1)
            sidx_acc = jnp.where(lane == b1 % 16, i1, sidx_acc)

            @pl.when(b1 % 16 == 15)
            def _():
                sidx_v[pl.ds(b1 - 15, 16)] = sidx_acc

            return sidx_acc

        lax.fori_loop(0, per_w // 2, gbody, jnp.zeros((16,), jnp.int32))

        cp_sb = pltpu.make_async_copy(sbias_hbm.at[sidx_v], sb_v, semm)
        cp_ie = pltpu.make_async_copy(iemb_hbm.at[iid_v],
                                      e0.at[pl.ds(0, per_w)], semm)
        cp_sb.start()
        cp_ie.start()
        cp_sb.wait()
        cp_ie.wait()

        def rbody(g, carry):
            raws = jnp.zeros((16,), jnp.float32)
            for k in range(16):
                r = row_dot(u_rows, 16 * g + k,
                            tuple(e0[16 * g + k, pl.ds(c * 16, 16)]
                                  for c in range(NACC)))
                raws = jnp.where(lane == k, r, raws)
            off = pl.ds(16 * g, 16)
            out_v[off] = (raws + AVG_RATING
                          + ub_v[off] + ib_v[off] + sb_v[off])
            return carry

        lax.fori_loop(0, per_w // 16, rbody, 0)
        pltpu.sync_copy(out_v, out_hbm.at[pl.ds(base, per_w)])

    return sc_call


def kernel(user_id, item_id, session_items_ids, session_items_daystamps,
           user_emb_table, item_emb_table, user_bias_table, item_bias_table,
           session_bias_table, user_decay_table):
    info = plsc.get_sparse_core_info()
    sid3 = session_items_ids.reshape(B, NCHUNK, LJ)
    sc_call = _build_sc_call(info.num_cores, info.num_subcores)
    return sc_call(user_id, item_id, sid3, session_items_daystamps,
                   user_emb_table, item_emb_table,
                   user_bias_table.reshape(USERS),
                   item_bias_table.reshape(ITEMS),
                   session_bias_table.reshape(SESSIONS),
                   user_decay_table.reshape(USERS))
